# baseline (speedup 1.0000x reference)
import jax
import jax.numpy as jnp
from jax.experimental import pallas as pl

N_LAYERS = 4
N_GRAPHS = 64


def _head_kernel(pooled_ref, wh_ref, bh_ref, out_ref):
    out_ref[...] = pooled_ref[...] @ wh_ref[...] + bh_ref[0, 0]


def kernel(x, edge_index, batch, W, att_src, att_dst, bias, W_head, b_head):
    N = x.shape[0]
    loop = jnp.arange(N, dtype=edge_index.dtype)
    src = jnp.concatenate([edge_index[0], loop])
    dst = jnp.concatenate([edge_index[1], loop])
    h = x
    for l in range(N_LAYERS):
        hW = h @ W[l]
        a_s = hW @ att_src[l]
        a_d = hW @ att_dst[l]
        e = a_s[src] + a_d[dst]
        e = jax.nn.leaky_relu(e, negative_slope=0.2)
        emax = jax.ops.segment_max(e, dst, num_segments=N)
        emax = jnp.where(jnp.isfinite(emax), emax, 0.0)
        ex = jnp.exp(e - emax[dst])
        denom = jax.ops.segment_sum(ex, dst, num_segments=N)
        alpha = ex / (denom[dst] + 1e-16)
        out = jax.ops.segment_sum(alpha[:, None] * hW[src], dst, num_segments=N)
        out = out + bias[l]
        if l < N_LAYERS - 1:
            out = jax.nn.relu(out)
        h = out
    sums = jax.ops.segment_sum(h, batch, num_segments=N_GRAPHS)
    counts = jax.ops.segment_sum(jnp.ones((N,), dtype=h.dtype), batch, num_segments=N_GRAPHS)
    pooled = sums / jnp.maximum(counts, 1.0)[:, None]
    return pl.pallas_call(
        _head_kernel,
        out_shape=jax.ShapeDtypeStruct((N_GRAPHS, 1), jnp.float32),
    )(pooled, W_head, b_head.reshape(1, 1))


# speedup vs baseline: 11.2055x; 11.2055x over previous
import dataclasses
import functools

import jax
import jax.numpy as jnp
from jax import lax
from jax.experimental import pallas as pl
from jax.experimental.pallas import tpu as pltpu
from jax.experimental.pallas import tpu_sc as plsc

N = 10000
E = 160000
E1 = E + N
D = 256
DH = 128
NLAYERS = 4
NGRAPHS = 64

NCORES = 2
NSUB = 16
LANES = 16
K = 64
CHUNKS = 167
E_PAD = NSUB * K * CHUNKS
EPB = K * CHUNKS
DROWS = 80


def _pre_body(h_ref, w_ref, asr_ref, adr_ref, hw2_ref, as_ref, ad_ref,
              amax_ref):
    hw = jnp.dot(h_ref[...], w_ref[...], preferred_element_type=jnp.float32)
    hw2_ref[0:N, :] = hw[:, 0:DH]
    hw2_ref[N : 2 * N, :] = hw[:, DH:D]
    a_s = jnp.sum(hw * asr_ref[...], axis=1)
    as_ref[...] = a_s
    ad_ref[...] = jnp.sum(hw * adr_ref[...], axis=1)
    amax_ref[...] = jnp.full((LANES,), jnp.max(a_s))


_tc_pre = pl.pallas_call(
    _pre_body,
    out_shape=[
        jax.ShapeDtypeStruct((2 * N, DH), jnp.float32),
        jax.ShapeDtypeStruct((N,), jnp.float32),
        jax.ShapeDtypeStruct((N,), jnp.float32),
        jax.ShapeDtypeStruct((LANES,), jnp.float32),
    ],
)


def _post_body(relu, o2_ref, b_ref, h_ref):
    h = jnp.concatenate([o2_ref[0:N, :], o2_ref[N : 2 * N, :]], axis=1)
    h = h + b_ref[...]
    if relu:
        h = jnp.maximum(h, 0.0)
    h_ref[...] = h


def _tc_post(relu):
    return pl.pallas_call(
        functools.partial(_post_body, relu),
        out_shape=jax.ShapeDtypeStruct((N, D), jnp.float32),
    )


def _pool_body(h_ref, b_ref, wh_ref, bh_ref, out_ref):
    gids = lax.broadcasted_iota(jnp.int32, (1, NGRAPHS), 1)
    onehot = (b_ref[...] == gids).astype(jnp.float32)
    sums = lax.dot_general(
        onehot, h_ref[...], (((0,), (0,)), ((), ())),
        preferred_element_type=jnp.float32)
    counts = jnp.sum(onehot, axis=0)
    pooled = sums / jnp.maximum(counts, 1.0)[:, None]
    out_ref[...] = (
        jnp.dot(pooled, wh_ref[...], preferred_element_type=jnp.float32)
        + bh_ref[0, 0]
    )


_tc_pool = pl.pallas_call(
    _pool_body,
    out_shape=jax.ShapeDtypeStruct((NGRAPHS, 1), jnp.float32),
)


_mesh = plsc.VectorSubcoreMesh(core_axis_name="c", subcore_axis_name="s")

_sc_params = pltpu.CompilerParams()
if "needs_layout_passes" in pltpu.CompilerParams.__dataclass_fields__:
    _sc_params = dataclasses.replace(_sc_params, needs_layout_passes=False)


@functools.partial(
    pl.kernel,
    mesh=_mesh,
    compiler_params=_sc_params,
    out_type=jax.ShapeDtypeStruct((2 * N, DH), jnp.float32),
    scratch_types=[
        pltpu.VMEM((N,), jnp.float32),
        pltpu.VMEM((N,), jnp.float32),
        pltpu.VMEM((LANES,), jnp.float32),
        pltpu.VMEM((K,), jnp.int32),
        pltpu.VMEM((K,), jnp.int32),
        pltpu.VMEM((1, K), jnp.int32),
        pltpu.VMEM((K,), jnp.float32),
        pltpu.VMEM((K, DH), jnp.float32),
        pltpu.VMEM((8, DH), jnp.float32),
        pltpu.VMEM((DROWS, DH), jnp.float32),
        pltpu.VMEM((1, DROWS), jnp.int32),
        pltpu.VMEM_SHARED((N, DH), jnp.float32),
        pltpu.VMEM_SHARED((DROWS, DH), jnp.float32),
        pltpu.SemaphoreType.DMA,
    ],
)
def _sc_edge(hw2_hbm, as_hbm, ad_hbm, amax_hbm, src_hbm, dst_hbm,
             out2_hbm,
             asv, adv, amaxv, srcv, srcg, dsti, exv, rows, zb,
             denv, dridx, acc_sh, den_sh, sem):
    cid = lax.axis_index("c")
    sid = lax.axis_index("s")
    zv = jnp.zeros((LANES,), jnp.float32)
    idx16 = lax.iota(jnp.int32, LANES)

    @pl.loop(0, 8)
    def _(r):
        for f in range(DH // LANES):
            zb[r, pl.ds(f * LANES, LANES)] = zv

    @pl.loop(0, DROWS)
    def _(r):
        for f in range(DH // LANES):
            denv[r, pl.ds(f * LANES, LANES)] = zv

    for t in range(DROWS // LANES):
        dridx[0, pl.ds(t * LANES, LANES)] = idx16 + t * LANES

    r0 = sid * 624

    @pl.loop(0, 78)
    def _(t):
        pltpu.sync_copy(zb, acc_sh.at[pl.ds(r0 + t * 8, 8)])

    @pl.when(sid == NSUB - 1)
    def _():
        for t in range(2):
            pltpu.sync_copy(zb, acc_sh.at[pl.ds(9984 + t * 8, 8)])

    @pl.when(sid == 0)
    def _():
        pltpu.sync_copy(denv, den_sh)

    pltpu.sync_copy(as_hbm, asv)
    pltpu.sync_copy(ad_hbm, adv)
    pltpu.sync_copy(amax_hbm, amaxv)
    amax = amaxv[pl.ds(0, LANES)]

    plsc.subcore_barrier()

    half = cid * N

    @pl.loop(0, CHUNKS)
    def _(ci):
        base = sid * EPB + ci * K
        pltpu.sync_copy(src_hbm.at[pl.ds(base, K)], srcv)
        pltpu.sync_copy(dst_hbm.at[pl.ds(base, K)], dsti.at[0])
        for j in range(K // LANES):
            sl = pl.ds(j * LANES, LANES)
            srcg[sl] = srcv[sl] + half
        gcp = pltpu.async_copy(hw2_hbm.at[srcg], rows, sem)

        for j in range(K // LANES):
            sl = pl.ds(j * LANES, LANES)
            dv = dsti[0, sl]
            a_sg = plsc.load_gather(asv, [srcv[sl]])
            a_dg = plsc.load_gather(adv, [dv])
            z = a_sg + a_dg
            e = jnp.where(z >= 0.0, z, 0.2 * z)
            c = amax + a_dg
            c = jnp.where(c >= 0.0, c, 0.2 * c)
            ex = jnp.exp(e - c)
            eid = idx16 + (base + j * LANES)
            ex = jnp.where(eid < E1, ex, 0.0)
            exv[sl] = ex
            drow = jax.lax.shift_right_logical(dv, 7)
            dcol = jnp.bitwise_and(dv, 127)
            for lane in range(LANES):
                plsc.addupdate_scatter(denv, [drow, dcol], ex,
                                       mask=idx16 == lane)

        gcp.wait()

        @pl.loop(0, K // LANES)
        def _(jj):
            exw = exv[pl.ds(jj * LANES, LANES)]
            row0 = jj * LANES
            for ei in range(LANES):
                w = jnp.full((LANES,), exw[ei])
                for f in range(DH // LANES):
                    sl = pl.ds(f * LANES, LANES)
                    rows[row0 + ei, sl] = rows[row0 + ei, sl] * w

        pltpu.sync_copy(rows, acc_sh.at[dsti.at[0]], add=True)

    pltpu.sync_copy(denv, den_sh.at[dridx.at[0]], add=True)
    plsc.subcore_barrier()
    pltpu.sync_copy(den_sh, denv)

    def _drain(rs, nrows):
        pltpu.sync_copy(acc_sh.at[pl.ds(rs, nrows)],
                        rows.at[pl.ds(0, nrows)])
        for b in range(nrows // LANES):
            nb = rs + b * LANES
            drow = jax.lax.shift_right_logical(nb, 7)
            dcol = jnp.bitwise_and(nb, 127)
            denw = denv[drow, pl.ds(dcol, LANES)]
            inv = 1.0 / (denw + 1e-16)
            for ei in range(LANES):
                w = jnp.full((LANES,), inv[ei])
                for f in range(DH // LANES):
                    sl = pl.ds(f * LANES, LANES)
                    rows[b * LANES + ei, sl] = rows[b * LANES + ei, sl] * w
        pltpu.sync_copy(rows.at[pl.ds(0, nrows)],
                        out2_hbm.at[pl.ds(half + rs, nrows)])

    @pl.loop(0, 9)
    def _(t):
        _drain(r0 + t * 64, 64)

    _drain(r0 + 576, 48)

    @pl.when(sid == NSUB - 1)
    def _():
        _drain(9984, 16)


def kernel(x, edge_index, batch, W, att_src, att_dst, bias, W_head, b_head):
    loop = jnp.arange(N, dtype=jnp.int32)
    pad = jnp.zeros((E_PAD - E1,), jnp.int32)
    src = jnp.concatenate([edge_index[0].astype(jnp.int32), loop, pad])
    dst = jnp.concatenate([edge_index[1].astype(jnp.int32), loop, pad])

    h = x
    for l in range(NLAYERS):
        hw2, a_s, a_d, amax = _tc_pre(h, W[l], att_src[l].reshape(1, D),
                                      att_dst[l].reshape(1, D))
        out2 = _sc_edge(hw2, a_s, a_d, amax, src, dst)
        h = _tc_post(l < NLAYERS - 1)(out2, bias[l].reshape(1, D))

    return _tc_pool(h, batch.astype(jnp.int32).reshape(N, 1),
                    W_head, b_head.reshape(1, 1))


# speedup vs baseline: 12.5637x; 1.1212x over previous
import dataclasses
import functools

import jax
import jax.numpy as jnp
from jax import lax
from jax.experimental import pallas as pl
from jax.experimental.pallas import tpu as pltpu
from jax.experimental.pallas import tpu_sc as plsc

N = 10000
E = 160000
E1 = E + N
D = 256
DH = 128
NLAYERS = 4
NGRAPHS = 64

NCORES = 2
NSUB = 16
LANES = 16
K = 128
CHUNKS = 84
E_PAD = NSUB * K * CHUNKS
EPB = K * CHUNKS
DROWS = 80


def _pre_body(h_ref, w_ref, asr_ref, adr_ref, hw2_ref, as_ref, ad_ref,
              amax_ref):
    hw = jnp.dot(h_ref[...], w_ref[...], preferred_element_type=jnp.float32)
    hw2_ref[0:N, :] = hw[:, 0:DH]
    hw2_ref[N : 2 * N, :] = hw[:, DH:D]
    a_s = jnp.sum(hw * asr_ref[...], axis=1)
    as_ref[...] = a_s
    ad_ref[...] = jnp.sum(hw * adr_ref[...], axis=1)
    amax_ref[...] = jnp.full((LANES,), jnp.max(a_s))


_tc_pre = pl.pallas_call(
    _pre_body,
    out_shape=[
        jax.ShapeDtypeStruct((2 * N, DH), jnp.float32),
        jax.ShapeDtypeStruct((N,), jnp.float32),
        jax.ShapeDtypeStruct((N,), jnp.float32),
        jax.ShapeDtypeStruct((LANES,), jnp.float32),
    ],
)


def _post_body(relu, o2_ref, b_ref, h_ref):
    h = jnp.concatenate([o2_ref[0:N, :], o2_ref[N : 2 * N, :]], axis=1)
    h = h + b_ref[...]
    if relu:
        h = jnp.maximum(h, 0.0)
    h_ref[...] = h


def _tc_post(relu):
    return pl.pallas_call(
        functools.partial(_post_body, relu),
        out_shape=jax.ShapeDtypeStruct((N, D), jnp.float32),
    )


def _pool_body(h_ref, b_ref, wh_ref, bh_ref, out_ref):
    gids = lax.broadcasted_iota(jnp.int32, (1, NGRAPHS), 1)
    onehot = (b_ref[...] == gids).astype(jnp.float32)
    sums = lax.dot_general(
        onehot, h_ref[...], (((0,), (0,)), ((), ())),
        preferred_element_type=jnp.float32)
    counts = jnp.sum(onehot, axis=0)
    pooled = sums / jnp.maximum(counts, 1.0)[:, None]
    out_ref[...] = (
        jnp.dot(pooled, wh_ref[...], preferred_element_type=jnp.float32)
        + bh_ref[0, 0]
    )


_tc_pool = pl.pallas_call(
    _pool_body,
    out_shape=jax.ShapeDtypeStruct((NGRAPHS, 1), jnp.float32),
)


_mesh = plsc.VectorSubcoreMesh(core_axis_name="c", subcore_axis_name="s")

_sc_params = pltpu.CompilerParams()
if "needs_layout_passes" in pltpu.CompilerParams.__dataclass_fields__:
    _sc_params = dataclasses.replace(_sc_params, needs_layout_passes=False)


@functools.partial(
    pl.kernel,
    mesh=_mesh,
    compiler_params=_sc_params,
    out_type=jax.ShapeDtypeStruct((2 * N, DH), jnp.float32),
    scratch_types=[
        pltpu.VMEM((N,), jnp.float32),
        pltpu.VMEM((N,), jnp.float32),
        pltpu.VMEM((LANES,), jnp.float32),
        pltpu.VMEM((K,), jnp.int32),
        pltpu.VMEM((K,), jnp.int32),
        pltpu.VMEM((1, K), jnp.int32),
        pltpu.VMEM((K,), jnp.float32),
        pltpu.VMEM((K, DH), jnp.float32),
        pltpu.VMEM((8, DH), jnp.float32),
        pltpu.VMEM((DROWS, DH), jnp.float32),
        pltpu.VMEM((1, DROWS), jnp.int32),
        pltpu.VMEM_SHARED((N, DH), jnp.float32),
        pltpu.VMEM_SHARED((DROWS, DH), jnp.float32),
        pltpu.SemaphoreType.DMA,
    ],
)
def _sc_edge(hw2_hbm, as_hbm, ad_hbm, amax_hbm, src_hbm, dst_hbm,
             out2_hbm,
             asv, adv, amaxv, srcv, srcg, dsti, exv, rows, zb,
             denv, dridx, acc_sh, den_sh, sem):
    cid = lax.axis_index("c")
    sid = lax.axis_index("s")
    zv = jnp.zeros((LANES,), jnp.float32)
    idx16 = lax.iota(jnp.int32, LANES)

    @pl.loop(0, 8)
    def _(r):
        for f in range(DH // LANES):
            zb[r, pl.ds(f * LANES, LANES)] = zv

    @pl.loop(0, DROWS)
    def _(r):
        for f in range(DH // LANES):
            denv[r, pl.ds(f * LANES, LANES)] = zv

    for t in range(DROWS // LANES):
        dridx[0, pl.ds(t * LANES, LANES)] = idx16 + t * LANES

    r0 = sid * 624

    @pl.loop(0, 78)
    def _(t):
        pltpu.sync_copy(zb, acc_sh.at[pl.ds(r0 + t * 8, 8)])

    @pl.when(sid == NSUB - 1)
    def _():
        for t in range(2):
            pltpu.sync_copy(zb, acc_sh.at[pl.ds(9984 + t * 8, 8)])

    @pl.when(sid == 0)
    def _():
        pltpu.sync_copy(denv, den_sh)

    pltpu.sync_copy(as_hbm, asv)
    pltpu.sync_copy(ad_hbm, adv)
    pltpu.sync_copy(amax_hbm, amaxv)
    amax = amaxv[pl.ds(0, LANES)]

    plsc.subcore_barrier()

    half = cid * N

    @pl.loop(0, CHUNKS)
    def _(ci):
        base = sid * EPB + ci * K
        pltpu.sync_copy(src_hbm.at[pl.ds(base, K)], srcv)
        pltpu.sync_copy(dst_hbm.at[pl.ds(base, K)], dsti.at[0])
        for j in range(K // LANES):
            sl = pl.ds(j * LANES, LANES)
            srcg[sl] = srcv[sl] + half
        gcp = pltpu.async_copy(hw2_hbm.at[srcg], rows, sem)

        for j in range(K // LANES):
            sl = pl.ds(j * LANES, LANES)
            dv = dsti[0, sl]
            a_sg = plsc.load_gather(asv, [srcv[sl]])
            a_dg = plsc.load_gather(adv, [dv])
            z = a_sg + a_dg
            e = jnp.where(z >= 0.0, z, 0.2 * z)
            c = amax + a_dg
            c = jnp.where(c >= 0.0, c, 0.2 * c)
            ex = jnp.exp(e - c)
            eid = idx16 + (base + j * LANES)
            ex = jnp.where(eid < E1, ex, 0.0)
            exv[sl] = ex
            drow = jax.lax.shift_right_logical(dv, 7)
            dcol = jnp.bitwise_and(dv, 127)
            for lane in range(LANES):
                plsc.addupdate_scatter(denv, [drow, dcol], ex,
                                       mask=idx16 == lane)

        gcp.wait()

        @pl.loop(0, K // LANES)
        def _(jj):
            exw = exv[pl.ds(jj * LANES, LANES)]
            row0 = jj * LANES
            for ei in range(LANES):
                w = jnp.full((LANES,), exw[ei])
                for f in range(DH // LANES):
                    sl = pl.ds(f * LANES, LANES)
                    rows[row0 + ei, sl] = rows[row0 + ei, sl] * w

        pltpu.sync_copy(rows, acc_sh.at[dsti.at[0]], add=True)

    pltpu.sync_copy(denv, den_sh.at[dridx.at[0]], add=True)
    plsc.subcore_barrier()
    pltpu.sync_copy(den_sh, denv)

    def _drain(rs, nrows):
        pltpu.sync_copy(acc_sh.at[pl.ds(rs, nrows)],
                        rows.at[pl.ds(0, nrows)])
        for b in range(nrows // LANES):
            nb = rs + b * LANES
            drow = jax.lax.shift_right_logical(nb, 7)
            dcol = jnp.bitwise_and(nb, 127)
            denw = denv[drow, pl.ds(dcol, LANES)]
            inv = 1.0 / (denw + 1e-16)
            for ei in range(LANES):
                w = jnp.full((LANES,), inv[ei])
                for f in range(DH // LANES):
                    sl = pl.ds(f * LANES, LANES)
                    rows[b * LANES + ei, sl] = rows[b * LANES + ei, sl] * w
        pltpu.sync_copy(rows.at[pl.ds(0, nrows)],
                        out2_hbm.at[pl.ds(half + rs, nrows)])

    @pl.loop(0, 9)
    def _(t):
        _drain(r0 + t * 64, 64)

    _drain(r0 + 576, 48)

    @pl.when(sid == NSUB - 1)
    def _():
        _drain(9984, 16)


def kernel(x, edge_index, batch, W, att_src, att_dst, bias, W_head, b_head):
    loop = jnp.arange(N, dtype=jnp.int32)
    pad = jnp.zeros((E_PAD - E1,), jnp.int32)
    src = jnp.concatenate([edge_index[0].astype(jnp.int32), loop, pad])
    dst = jnp.concatenate([edge_index[1].astype(jnp.int32), loop, pad])

    h = x
    for l in range(NLAYERS):
        hw2, a_s, a_d, amax = _tc_pre(h, W[l], att_src[l].reshape(1, D),
                                      att_dst[l].reshape(1, D))
        out2 = _sc_edge(hw2, a_s, a_d, amax, src, dst)
        h = _tc_post(l < NLAYERS - 1)(out2, bias[l].reshape(1, D))

    return _tc_pool(h, batch.astype(jnp.int32).reshape(N, 1),
                    W_head, b_head.reshape(1, 1))
